# Initial kernel scaffold; baseline (speedup 1.0000x reference)
import jax
import jax.numpy as jnp
from jax.experimental import pallas as pl


def kernel(x, adj, W_att, a_att, Wi_f, Wh_f, b_f, Wi_b, Wh_b, b_b, W_out, a_out):
    raise NotImplementedError("write your pallas kernel here")


# speedup vs baseline: 10.7702x; 10.7702x over previous
import functools

import jax
import jax.numpy as jnp
from jax.experimental import pallas as pl
from jax.experimental.pallas import tpu as pltpu

_ALPHA = 0.2
_NEG = -9e15


def _pick_tile(n, target):
    best = None
    for d in range(8, min(n, target) + 1, 8):
        if n % d == 0:
            best = d
    return best if best is not None else n


def _leaky(v):
    return jnp.where(v >= 0, v, _ALPHA * v)


def _elu(v):
    return jnp.where(v > 0, v, jnp.exp(v) - 1.0)


def _k1_body(nheads, x_ref, watt_ref, aatt_ref, wh_ref, f1_ref, f2_ref):
    xb = x_ref[...]
    for h in range(nheads):
        whh = jnp.dot(xb, watt_ref[h], preferred_element_type=jnp.float32)
        wh_ref[h] = whh
        a = aatt_ref[h]
        hdim = whh.shape[1]
        f1 = jnp.dot(whh, a[:hdim, :], preferred_element_type=jnp.float32)
        f2 = jnp.dot(whh, a[hdim:, :], preferred_element_type=jnp.float32)
        f1_ref[:, h:h + 1] = f1
        f2_ref[:, h:h + 1] = f2
    return


def _k2_body(nheads, n, tc, adj_ref, f1_ref, f2_ref, wh_ref, out_ref,
             acc_ref, m_ref, l_ref):
    c = pl.program_id(1)
    nc = pl.num_programs(1)

    @pl.when(c == 0)
    def _init():
        m_ref[...] = jnp.full_like(m_ref, -jnp.inf)
        l_ref[...] = jnp.zeros_like(l_ref)
        acc_ref[...] = jnp.zeros_like(acc_ref)

    base = c * tc
    col_ok_row = base + jax.lax.broadcasted_iota(jnp.int32, (1, tc), 1) < n
    col_ok_col = base + jax.lax.broadcasted_iota(jnp.int32, (tc, 1), 0) < n
    mask = (adj_ref[...] > 0) & col_ok_row
    for h in range(nheads):
        f1 = f1_ref[:, h:h + 1]
        f2 = jnp.reshape(f2_ref[:, h:h + 1], (1, tc))
        e = _leaky(f1 + f2)
        e = jnp.where(mask, e, _NEG)
        e = jnp.where(col_ok_row, e, -jnp.inf)
        m_old = m_ref[:, h:h + 1]
        m_new = jnp.maximum(m_old, jnp.max(e, axis=1, keepdims=True))
        p = jnp.exp(e - m_new)
        scale = jnp.exp(m_old - m_new)
        l_ref[:, h:h + 1] = l_ref[:, h:h + 1] * scale + jnp.sum(
            p, axis=1, keepdims=True)
        whh = jnp.where(col_ok_col, wh_ref[h], 0.0)
        pv = jnp.dot(p, whh, preferred_element_type=jnp.float32)
        acc_ref[h] = acc_ref[h] * scale + pv
        m_ref[:, h:h + 1] = m_new

    @pl.when(c == nc - 1)
    def _fin():
        for h in range(nheads):
            out_ref[h] = acc_ref[h] / l_ref[:, h:h + 1]


def _k3_body(nheads, ts, hf_in_ref, hb_in_ref, wif_ref, whf_ref, bf_ref,
             wib_ref, whb_ref, bb_ref, hf_out_ref, hb_out_ref,
             xif_ref, xib_ref, state_ref):
    k = pl.program_id(0)
    hdim = whf_ref.shape[0]
    nh = nheads

    @pl.when(k == 0)
    def _init():
        state_ref[...] = jnp.zeros_like(state_ref)

    hf_in = jnp.reshape(hf_in_ref[...], (nh * ts, hdim))
    hb_in = jnp.reshape(hb_in_ref[...], (nh * ts, hdim))
    xif_ref[...] = jnp.reshape(
        jnp.dot(hf_in, wif_ref[...], preferred_element_type=jnp.float32)
        + bf_ref[...], (nh, ts, 4 * hdim))
    xib_ref[...] = jnp.reshape(
        jnp.dot(hb_in, wib_ref[...], preferred_element_type=jnp.float32)
        + bb_ref[...], (nh, ts, 4 * hdim))

    whf = whf_ref[...]
    whb = whb_ref[...]
    hf = state_ref[0:nh, :]
    cf = state_ref[nh:2 * nh, :]
    hb = state_ref[2 * nh:3 * nh, :]
    cb = state_ref[3 * nh:4 * nh, :]

    def step(t, carry):
        hf, cf, hb, cb = carry
        tb = ts - 1 - t
        zf = jnp.reshape(xif_ref[:, pl.ds(t, 1), :], (nh, 4 * hdim)) + jnp.dot(
            hf, whf, preferred_element_type=jnp.float32)
        zb = jnp.reshape(xib_ref[:, pl.ds(tb, 1), :], (nh, 4 * hdim)) + jnp.dot(
            hb, whb, preferred_element_type=jnp.float32)
        i_f = jax.nn.sigmoid(zf[:, :hdim])
        f_f = jax.nn.sigmoid(zf[:, hdim:2 * hdim])
        g_f = jnp.tanh(zf[:, 2 * hdim:3 * hdim])
        o_f = jax.nn.sigmoid(zf[:, 3 * hdim:])
        cf = f_f * cf + i_f * g_f
        hf = o_f * jnp.tanh(cf)
        i_b = jax.nn.sigmoid(zb[:, :hdim])
        f_b = jax.nn.sigmoid(zb[:, hdim:2 * hdim])
        g_b = jnp.tanh(zb[:, 2 * hdim:3 * hdim])
        o_b = jax.nn.sigmoid(zb[:, 3 * hdim:])
        cb = f_b * cb + i_b * g_b
        hb = o_b * jnp.tanh(cb)
        hf_out_ref[:, pl.ds(t, 1), :] = jnp.reshape(hf, (nh, 1, hdim))
        hb_out_ref[:, pl.ds(tb, 1), :] = jnp.reshape(hb, (nh, 1, hdim))
        return hf, cf, hb, cb

    hf, cf, hb, cb = jax.lax.fori_loop(0, ts, step, (hf, cf, hb, cb))
    state_ref[0:nh, :] = hf
    state_ref[nh:2 * nh, :] = cf
    state_ref[2 * nh:3 * nh, :] = hb
    state_ref[3 * nh:4 * nh, :] = cb


def _k4_body(nheads, hf_ref, hb_ref, wout_ref, who_ref):
    hdim = hf_ref.shape[2]
    tr = hf_ref.shape[1]
    acc = jnp.zeros((tr, wout_ref.shape[1]), jnp.float32)
    for h in range(nheads):
        y = _elu(hf_ref[h] + hb_ref[h])
        acc = acc + jnp.dot(y, wout_ref[h * hdim:(h + 1) * hdim, :],
                            preferred_element_type=jnp.float32)
    who_ref[...] = acc


def _k5_body(n, tc, adj_ref, whor_ref, whoc_ref, aout_ref, out_ref,
             acc_ref, m_ref, l_ref):
    c = pl.program_id(1)
    nc = pl.num_programs(1)

    @pl.when(c == 0)
    def _init():
        m_ref[...] = jnp.full_like(m_ref, -jnp.inf)
        l_ref[...] = jnp.zeros_like(l_ref)
        acc_ref[...] = jnp.zeros_like(acc_ref)

    base = c * tc
    col_ok_row = base + jax.lax.broadcasted_iota(jnp.int32, (1, tc), 1) < n
    col_ok_col = base + jax.lax.broadcasted_iota(jnp.int32, (tc, 1), 0) < n
    a = aout_ref[...]
    a0 = a[0, 0]
    a1 = a[0, 1]
    whoc = jnp.where(col_ok_col, whoc_ref[...], 0.0)
    whoc_row = jnp.reshape(whoc, (1, whoc.shape[0]))
    f1 = whor_ref[...] * a0
    f2 = whoc_row * a1
    e = _leaky(f1 + f2)
    e = jnp.where((adj_ref[...] > 0) & col_ok_row, e, _NEG)
    e = jnp.where(col_ok_row, e, -jnp.inf)
    m_old = m_ref[...]
    m_new = jnp.maximum(m_old, jnp.max(e, axis=1, keepdims=True))
    p = jnp.exp(e - m_new)
    scale = jnp.exp(m_old - m_new)
    l_ref[...] = l_ref[...] * scale + jnp.sum(p, axis=1, keepdims=True)
    acc_ref[...] = acc_ref[...] * scale + jnp.sum(
        p * whoc_row, axis=1, keepdims=True)
    m_ref[...] = m_new

    @pl.when(c == nc - 1)
    def _fin():
        out_ref[...] = _elu(acc_ref[...] / l_ref[...])


def kernel(x, adj, W_att, a_att, Wi_f, Wh_f, b_f, Wi_b, Wh_b, b_b, W_out,
           a_out):
    n, nfeat = x.shape
    nheads, _, hdim = W_att.shape
    outd = W_out.shape[1]
    f32 = jnp.float32

    trp = _pick_tile(n, 2000)
    tr = _pick_tile(n, 400)
    tc = min(2048, ((n + 127) // 128) * 128)
    ts = _pick_tile(n, 1000)
    ncol = pl.cdiv(n, tc)

    wh, f1, f2 = pl.pallas_call(
        functools.partial(_k1_body, nheads),
        grid=(n // trp,),
        in_specs=[
            pl.BlockSpec((trp, nfeat), lambda r: (r, 0)),
            pl.BlockSpec((nheads, nfeat, hdim), lambda r: (0, 0, 0)),
            pl.BlockSpec((nheads, 2 * hdim, 1), lambda r: (0, 0, 0)),
        ],
        out_specs=[
            pl.BlockSpec((nheads, trp, hdim), lambda r: (0, r, 0)),
            pl.BlockSpec((trp, nheads), lambda r: (r, 0)),
            pl.BlockSpec((trp, nheads), lambda r: (r, 0)),
        ],
        out_shape=[
            jax.ShapeDtypeStruct((nheads, n, hdim), f32),
            jax.ShapeDtypeStruct((n, nheads), f32),
            jax.ShapeDtypeStruct((n, nheads), f32),
        ],
    )(x, W_att, a_att)

    hagg = pl.pallas_call(
        functools.partial(_k2_body, nheads, n, tc),
        grid=(n // tr, ncol),
        in_specs=[
            pl.BlockSpec((tr, tc), lambda r, c: (r, c)),
            pl.BlockSpec((tr, nheads), lambda r, c: (r, 0)),
            pl.BlockSpec((tc, nheads), lambda r, c: (c, 0)),
            pl.BlockSpec((nheads, tc, hdim), lambda r, c: (0, c, 0)),
        ],
        out_specs=pl.BlockSpec((nheads, tr, hdim), lambda r, c: (0, r, 0)),
        out_shape=jax.ShapeDtypeStruct((nheads, n, hdim), f32),
        scratch_shapes=[
            pltpu.VMEM((nheads, tr, hdim), f32),
            pltpu.VMEM((tr, nheads), f32),
            pltpu.VMEM((tr, nheads), f32),
        ],
    )(adj, f1, f2, wh)

    nk = n // ts
    bf2 = jnp.reshape(b_f, (1, 4 * hdim))
    bb2 = jnp.reshape(b_b, (1, 4 * hdim))
    hf, hb = pl.pallas_call(
        functools.partial(_k3_body, nheads, ts),
        grid=(nk,),
        in_specs=[
            pl.BlockSpec((nheads, ts, hdim), lambda k: (0, k, 0)),
            pl.BlockSpec((nheads, ts, hdim), lambda k: (0, nk - 1 - k, 0)),
            pl.BlockSpec((hdim, 4 * hdim), lambda k: (0, 0)),
            pl.BlockSpec((hdim, 4 * hdim), lambda k: (0, 0)),
            pl.BlockSpec((1, 4 * hdim), lambda k: (0, 0)),
            pl.BlockSpec((hdim, 4 * hdim), lambda k: (0, 0)),
            pl.BlockSpec((hdim, 4 * hdim), lambda k: (0, 0)),
            pl.BlockSpec((1, 4 * hdim), lambda k: (0, 0)),
        ],
        out_specs=[
            pl.BlockSpec((nheads, ts, hdim), lambda k: (0, k, 0)),
            pl.BlockSpec((nheads, ts, hdim), lambda k: (0, nk - 1 - k, 0)),
        ],
        out_shape=[
            jax.ShapeDtypeStruct((nheads, n, hdim), f32),
            jax.ShapeDtypeStruct((nheads, n, hdim), f32),
        ],
        scratch_shapes=[
            pltpu.VMEM((nheads, ts, 4 * hdim), f32),
            pltpu.VMEM((nheads, ts, 4 * hdim), f32),
            pltpu.VMEM((4 * nheads, hdim), f32),
        ],
    )(hagg, hagg, Wi_f, Wh_f, bf2, Wi_b, Wh_b, bb2)

    who = pl.pallas_call(
        functools.partial(_k4_body, nheads),
        grid=(n // trp,),
        in_specs=[
            pl.BlockSpec((nheads, trp, hdim), lambda r: (0, r, 0)),
            pl.BlockSpec((nheads, trp, hdim), lambda r: (0, r, 0)),
            pl.BlockSpec((nheads * hdim, outd), lambda r: (0, 0)),
        ],
        out_specs=pl.BlockSpec((trp, outd), lambda r: (r, 0)),
        out_shape=jax.ShapeDtypeStruct((n, outd), f32),
    )(hf, hb, W_out)

    aout2 = jnp.reshape(a_out, (1, 2 * outd))
    out = pl.pallas_call(
        functools.partial(_k5_body, n, tc),
        grid=(n // tr, ncol),
        in_specs=[
            pl.BlockSpec((tr, tc), lambda r, c: (r, c)),
            pl.BlockSpec((tr, outd), lambda r, c: (r, 0)),
            pl.BlockSpec((tc, outd), lambda r, c: (c, 0)),
            pl.BlockSpec((1, 2 * outd), lambda r, c: (0, 0)),
        ],
        out_specs=pl.BlockSpec((tr, outd), lambda r, c: (r, 0)),
        out_shape=jax.ShapeDtypeStruct((n, outd), f32),
        scratch_shapes=[
            pltpu.VMEM((tr, outd), f32),
            pltpu.VMEM((tr, outd), f32),
            pltpu.VMEM((tr, outd), f32),
        ],
    )(adj, who, who, aout2)

    return out


# speedup vs baseline: 11.2227x; 1.0420x over previous
import functools

import jax
import jax.numpy as jnp
from jax.experimental import pallas as pl
from jax.experimental.pallas import tpu as pltpu

_ALPHA = 0.2
_NEG = -9e15


def _pick_tile(n, target):
    best = None
    for d in range(8, min(n, target) + 1, 8):
        if n % d == 0:
            best = d
    return best if best is not None else n


def _leaky(v):
    return jnp.maximum(v, _ALPHA * v)


def _elu(v):
    return jnp.where(v > 0, v, jnp.exp(v) - 1.0)


def _k1_body(nheads, x_ref, watt_ref, aatt_ref, wh_ref, f1_ref, f2_ref):
    xb = x_ref[...]
    for h in range(nheads):
        whh = jnp.dot(xb, watt_ref[h], preferred_element_type=jnp.float32)
        wh_ref[h] = whh
        a = aatt_ref[h]
        hdim = whh.shape[1]
        f1 = jnp.dot(whh, a[:hdim, :], preferred_element_type=jnp.float32)
        f2 = jnp.dot(whh, a[hdim:, :], preferred_element_type=jnp.float32)
        f1_ref[:, h:h + 1] = f1
        f2_ref[:, h:h + 1] = f2
    return


def _k2_body(nheads, n, tc, adj_ref, f1_ref, f2_ref, wh_ref, out_ref,
             acc_ref, m_ref, l_ref):
    c = pl.program_id(1)
    nc = pl.num_programs(1)

    @pl.when(c == 0)
    def _init():
        m_ref[...] = jnp.full_like(m_ref, -jnp.inf)
        l_ref[...] = jnp.zeros_like(l_ref)
        acc_ref[...] = jnp.zeros_like(acc_ref)

    base = c * tc
    col_ok_row = base + jax.lax.broadcasted_iota(jnp.int32, (1, tc), 1) < n
    mask = (adj_ref[...] > 0) & col_ok_row
    neg_row = jnp.where(col_ok_row, _NEG, -jnp.inf)
    col_ok_3d = base + jax.lax.broadcasted_iota(jnp.int32, (1, tc, 1), 1) < n
    whs = jnp.where(col_ok_3d, wh_ref[...], 0.0)
    for h in range(nheads):
        f1 = f1_ref[:, h:h + 1]
        f2 = jnp.reshape(f2_ref[:, h:h + 1], (1, tc))
        e = jnp.where(mask, _leaky(f1 + f2), neg_row)
        m_old = m_ref[:, h:h + 1]
        m_new = jnp.maximum(m_old, jnp.max(e, axis=1, keepdims=True))
        p = jnp.exp(e - m_new)
        scale = jnp.exp(m_old - m_new)
        l_ref[:, h:h + 1] = l_ref[:, h:h + 1] * scale + jnp.sum(
            p, axis=1, keepdims=True)
        pv = jnp.dot(p, whs[h], preferred_element_type=jnp.float32)
        acc_ref[h] = acc_ref[h] * scale + pv
        m_ref[:, h:h + 1] = m_new

    @pl.when(c == nc - 1)
    def _fin():
        for h in range(nheads):
            out_ref[h] = acc_ref[h] / l_ref[:, h:h + 1]


def _k3_body(nheads, ts, hf_in_ref, hb_in_ref, wif_ref, whf_ref, bf_ref,
             wib_ref, whb_ref, bb_ref, hf_out_ref, hb_out_ref,
             xif_ref, xib_ref, state_ref):
    k = pl.program_id(0)
    hdim = whf_ref.shape[0]
    nh = nheads

    @pl.when(k == 0)
    def _init():
        state_ref[...] = jnp.zeros_like(state_ref)

    hf_in = jnp.reshape(hf_in_ref[...], (nh * ts, hdim))
    hb_in = jnp.reshape(hb_in_ref[...], (nh * ts, hdim))
    xif_ref[...] = jnp.reshape(
        jnp.dot(hf_in, wif_ref[...], preferred_element_type=jnp.float32)
        + bf_ref[...], (nh, ts, 4 * hdim))
    xib_ref[...] = jnp.reshape(
        jnp.dot(hb_in, wib_ref[...], preferred_element_type=jnp.float32)
        + bb_ref[...], (nh, ts, 4 * hdim))

    whf = whf_ref[...]
    whb = whb_ref[...]
    hf = state_ref[0:nh, :]
    cf = state_ref[nh:2 * nh, :]
    hb = state_ref[2 * nh:3 * nh, :]
    cb = state_ref[3 * nh:4 * nh, :]

    def step(t, carry):
        hf, cf, hb, cb = carry
        tb = ts - 1 - t
        zf = jnp.reshape(xif_ref[:, pl.ds(t, 1), :], (nh, 4 * hdim)) + jnp.dot(
            hf, whf, preferred_element_type=jnp.float32)
        zb = jnp.reshape(xib_ref[:, pl.ds(tb, 1), :], (nh, 4 * hdim)) + jnp.dot(
            hb, whb, preferred_element_type=jnp.float32)
        i_f = jax.nn.sigmoid(zf[:, :hdim])
        f_f = jax.nn.sigmoid(zf[:, hdim:2 * hdim])
        g_f = jnp.tanh(zf[:, 2 * hdim:3 * hdim])
        o_f = jax.nn.sigmoid(zf[:, 3 * hdim:])
        cf = f_f * cf + i_f * g_f
        hf = o_f * jnp.tanh(cf)
        i_b = jax.nn.sigmoid(zb[:, :hdim])
        f_b = jax.nn.sigmoid(zb[:, hdim:2 * hdim])
        g_b = jnp.tanh(zb[:, 2 * hdim:3 * hdim])
        o_b = jax.nn.sigmoid(zb[:, 3 * hdim:])
        cb = f_b * cb + i_b * g_b
        hb = o_b * jnp.tanh(cb)
        hf_out_ref[:, pl.ds(t, 1), :] = jnp.reshape(hf, (nh, 1, hdim))
        hb_out_ref[:, pl.ds(tb, 1), :] = jnp.reshape(hb, (nh, 1, hdim))
        return hf, cf, hb, cb

    hf, cf, hb, cb = jax.lax.fori_loop(0, ts, step, (hf, cf, hb, cb))
    state_ref[0:nh, :] = hf
    state_ref[nh:2 * nh, :] = cf
    state_ref[2 * nh:3 * nh, :] = hb
    state_ref[3 * nh:4 * nh, :] = cb


def _k4_body(nheads, hf_ref, hb_ref, wout_ref, who_ref):
    hdim = hf_ref.shape[2]
    tr = hf_ref.shape[1]
    acc = jnp.zeros((tr, wout_ref.shape[1]), jnp.float32)
    for h in range(nheads):
        y = _elu(hf_ref[h] + hb_ref[h])
        acc = acc + jnp.dot(y, wout_ref[h * hdim:(h + 1) * hdim, :],
                            preferred_element_type=jnp.float32)
    who_ref[...] = acc


def _k5_body(n, tc, adj_ref, whor_ref, whoc_ref, aout_ref, out_ref,
             acc_ref, m_ref, l_ref):
    c = pl.program_id(1)
    nc = pl.num_programs(1)

    @pl.when(c == 0)
    def _init():
        m_ref[...] = jnp.full_like(m_ref, -jnp.inf)
        l_ref[...] = jnp.zeros_like(l_ref)
        acc_ref[...] = jnp.zeros_like(acc_ref)

    base = c * tc
    col_ok_row = base + jax.lax.broadcasted_iota(jnp.int32, (1, tc), 1) < n
    col_ok_col = base + jax.lax.broadcasted_iota(jnp.int32, (tc, 1), 0) < n
    a = aout_ref[...]
    a0 = a[0, 0]
    a1 = a[0, 1]
    whoc = jnp.where(col_ok_col, whoc_ref[...], 0.0)
    whoc_row = jnp.reshape(whoc, (1, whoc.shape[0]))
    f1 = whor_ref[...] * a0
    f2 = whoc_row * a1
    neg_row = jnp.where(col_ok_row, _NEG, -jnp.inf)
    mask = (adj_ref[...] > 0) & col_ok_row
    e = jnp.where(mask, _leaky(f1 + f2), neg_row)
    m_old = m_ref[...]
    m_new = jnp.maximum(m_old, jnp.max(e, axis=1, keepdims=True))
    p = jnp.exp(e - m_new)
    scale = jnp.exp(m_old - m_new)
    l_ref[...] = l_ref[...] * scale + jnp.sum(p, axis=1, keepdims=True)
    acc_ref[...] = acc_ref[...] * scale + jnp.sum(
        p * whoc_row, axis=1, keepdims=True)
    m_ref[...] = m_new

    @pl.when(c == nc - 1)
    def _fin():
        out_ref[...] = _elu(acc_ref[...] / l_ref[...])


def kernel(x, adj, W_att, a_att, Wi_f, Wh_f, b_f, Wi_b, Wh_b, b_b, W_out,
           a_out):
    n, nfeat = x.shape
    nheads, _, hdim = W_att.shape
    outd = W_out.shape[1]
    f32 = jnp.float32

    trp = _pick_tile(n, 2000)
    tr = _pick_tile(n, 400)
    tc = min(2048, ((n + 127) // 128) * 128)
    ts = _pick_tile(n, 1000)
    ncol = pl.cdiv(n, tc)

    wh, f1, f2 = pl.pallas_call(
        functools.partial(_k1_body, nheads),
        grid=(n // trp,),
        in_specs=[
            pl.BlockSpec((trp, nfeat), lambda r: (r, 0)),
            pl.BlockSpec((nheads, nfeat, hdim), lambda r: (0, 0, 0)),
            pl.BlockSpec((nheads, 2 * hdim, 1), lambda r: (0, 0, 0)),
        ],
        out_specs=[
            pl.BlockSpec((nheads, trp, hdim), lambda r: (0, r, 0)),
            pl.BlockSpec((trp, nheads), lambda r: (r, 0)),
            pl.BlockSpec((trp, nheads), lambda r: (r, 0)),
        ],
        out_shape=[
            jax.ShapeDtypeStruct((nheads, n, hdim), f32),
            jax.ShapeDtypeStruct((n, nheads), f32),
            jax.ShapeDtypeStruct((n, nheads), f32),
        ],
    )(x, W_att, a_att)

    hagg = pl.pallas_call(
        functools.partial(_k2_body, nheads, n, tc),
        grid=(n // tr, ncol),
        in_specs=[
            pl.BlockSpec((tr, tc), lambda r, c: (r, c)),
            pl.BlockSpec((tr, nheads), lambda r, c: (r, 0)),
            pl.BlockSpec((tc, nheads), lambda r, c: (c, 0)),
            pl.BlockSpec((nheads, tc, hdim), lambda r, c: (0, c, 0)),
        ],
        out_specs=pl.BlockSpec((nheads, tr, hdim), lambda r, c: (0, r, 0)),
        out_shape=jax.ShapeDtypeStruct((nheads, n, hdim), f32),
        scratch_shapes=[
            pltpu.VMEM((nheads, tr, hdim), f32),
            pltpu.VMEM((tr, nheads), f32),
            pltpu.VMEM((tr, nheads), f32),
        ],
    )(adj, f1, f2, wh)

    nk = n // ts
    bf2 = jnp.reshape(b_f, (1, 4 * hdim))
    bb2 = jnp.reshape(b_b, (1, 4 * hdim))
    hf, hb = pl.pallas_call(
        functools.partial(_k3_body, nheads, ts),
        grid=(nk,),
        in_specs=[
            pl.BlockSpec((nheads, ts, hdim), lambda k: (0, k, 0)),
            pl.BlockSpec((nheads, ts, hdim), lambda k: (0, nk - 1 - k, 0)),
            pl.BlockSpec((hdim, 4 * hdim), lambda k: (0, 0)),
            pl.BlockSpec((hdim, 4 * hdim), lambda k: (0, 0)),
            pl.BlockSpec((1, 4 * hdim), lambda k: (0, 0)),
            pl.BlockSpec((hdim, 4 * hdim), lambda k: (0, 0)),
            pl.BlockSpec((hdim, 4 * hdim), lambda k: (0, 0)),
            pl.BlockSpec((1, 4 * hdim), lambda k: (0, 0)),
        ],
        out_specs=[
            pl.BlockSpec((nheads, ts, hdim), lambda k: (0, k, 0)),
            pl.BlockSpec((nheads, ts, hdim), lambda k: (0, nk - 1 - k, 0)),
        ],
        out_shape=[
            jax.ShapeDtypeStruct((nheads, n, hdim), f32),
            jax.ShapeDtypeStruct((nheads, n, hdim), f32),
        ],
        scratch_shapes=[
            pltpu.VMEM((nheads, ts, 4 * hdim), f32),
            pltpu.VMEM((nheads, ts, 4 * hdim), f32),
            pltpu.VMEM((4 * nheads, hdim), f32),
        ],
    )(hagg, hagg, Wi_f, Wh_f, bf2, Wi_b, Wh_b, bb2)

    who = pl.pallas_call(
        functools.partial(_k4_body, nheads),
        grid=(n // trp,),
        in_specs=[
            pl.BlockSpec((nheads, trp, hdim), lambda r: (0, r, 0)),
            pl.BlockSpec((nheads, trp, hdim), lambda r: (0, r, 0)),
            pl.BlockSpec((nheads * hdim, outd), lambda r: (0, 0)),
        ],
        out_specs=pl.BlockSpec((trp, outd), lambda r: (r, 0)),
        out_shape=jax.ShapeDtypeStruct((n, outd), f32),
    )(hf, hb, W_out)

    aout2 = jnp.reshape(a_out, (1, 2 * outd))
    out = pl.pallas_call(
        functools.partial(_k5_body, n, tc),
        grid=(n // tr, ncol),
        in_specs=[
            pl.BlockSpec((tr, tc), lambda r, c: (r, c)),
            pl.BlockSpec((tr, outd), lambda r, c: (r, 0)),
            pl.BlockSpec((tc, outd), lambda r, c: (c, 0)),
            pl.BlockSpec((1, 2 * outd), lambda r, c: (0, 0)),
        ],
        out_specs=pl.BlockSpec((tr, outd), lambda r, c: (r, 0)),
        out_shape=jax.ShapeDtypeStruct((n, outd), f32),
        scratch_shapes=[
            pltpu.VMEM((tr, outd), f32),
            pltpu.VMEM((tr, outd), f32),
            pltpu.VMEM((tr, outd), f32),
        ],
    )(adj, who, who, aout2)

    return out


# speedup vs baseline: 11.4745x; 1.0224x over previous
import functools

import jax
import jax.numpy as jnp
from jax.experimental import pallas as pl
from jax.experimental.pallas import tpu as pltpu

_ALPHA = 0.2


def _pick_tile(n, target):
    best = None
    for d in range(8, min(n, target) + 1, 8):
        if n % d == 0:
            best = d
    return best if best is not None else n


def _leaky(v):
    return jnp.maximum(v, _ALPHA * v)


def _elu(v):
    return jnp.where(v > 0, v, jnp.exp(v) - 1.0)


def _k1_body(nheads, x_ref, watt_ref, aatt_ref,
             wh_ref, f1_ref, f2_ref, fmax_ref, wmean_ref,
             fmx_ref, wsum_ref):
    r = pl.program_id(0)
    nr = pl.num_programs(0)
    n_rows = x_ref.shape[0] * nr

    @pl.when(r == 0)
    def _init():
        fmx_ref[...] = jnp.full_like(fmx_ref, -jnp.inf)
        wsum_ref[...] = jnp.zeros_like(wsum_ref)

    xb = x_ref[...]
    for h in range(nheads):
        whh = jnp.dot(xb, watt_ref[h], preferred_element_type=jnp.float32)
        wh_ref[h] = whh
        a = aatt_ref[h]
        hdim = whh.shape[1]
        f1 = jnp.dot(whh, a[:hdim, :], preferred_element_type=jnp.float32)
        f2 = jnp.dot(whh, a[hdim:, :], preferred_element_type=jnp.float32)
        f1_ref[:, h:h + 1] = f1
        f2_ref[:, h:h + 1] = f2
        fmx_ref[:, h:h + 1] = jnp.maximum(
            fmx_ref[:, h:h + 1], jnp.max(f2, keepdims=True))
        wsum_ref[h:h + 1, :] = wsum_ref[h:h + 1, :] + jnp.sum(
            whh, axis=0, keepdims=True)

    @pl.when(r == nr - 1)
    def _fin():
        fmax_ref[...] = fmx_ref[...]
        wmean_ref[...] = wsum_ref[...] / n_rows


def _k2_body(nheads, n, tc, adj_ref, f1_ref, f2_ref, wh_ref, fmax_ref,
             wmean_ref, out_ref, acc_ref, l_ref):
    c = pl.program_id(1)
    nc = pl.num_programs(1)

    @pl.when(c == 0)
    def _init():
        l_ref[...] = jnp.zeros_like(l_ref)
        acc_ref[...] = jnp.zeros_like(acc_ref)

    base = c * tc
    col_ok_row = base + jax.lax.broadcasted_iota(jnp.int32, (1, tc), 1) < n
    mask = (adj_ref[...] > 0) & col_ok_row
    col_ok_3d = base + jax.lax.broadcasted_iota(jnp.int32, (1, tc, 1), 1) < n
    whs = jnp.where(col_ok_3d, wh_ref[...], 0.0)
    for h in range(nheads):
        f1 = f1_ref[:, h:h + 1]
        f2 = jnp.reshape(f2_ref[:, h:h + 1], (1, tc))
        m = _leaky(f1 + fmax_ref[:, h:h + 1])
        p = jnp.where(mask, jnp.exp(_leaky(f1 + f2) - m), 0.0)
        l_ref[:, h:h + 1] = l_ref[:, h:h + 1] + jnp.sum(
            p, axis=1, keepdims=True)
        acc_ref[h] = acc_ref[h] + jnp.dot(
            p, whs[h], preferred_element_type=jnp.float32)

    @pl.when(c == nc - 1)
    def _fin():
        for h in range(nheads):
            l = l_ref[:, h:h + 1]
            out_ref[h] = jnp.where(l > 0,
                                   acc_ref[h] / jnp.maximum(l, 1e-37),
                                   wmean_ref[h:h + 1, :])


def _k3_body(nheads, ts, hf_in_ref, hb_in_ref, wif_ref, whf_ref, bf_ref,
             wib_ref, whb_ref, bb_ref, hf_out_ref, hb_out_ref,
             xif_ref, xib_ref, state_ref):
    k = pl.program_id(0)
    hdim = whf_ref.shape[0]
    nh = nheads

    @pl.when(k == 0)
    def _init():
        state_ref[...] = jnp.zeros_like(state_ref)

    hf_in = jnp.reshape(hf_in_ref[...], (nh * ts, hdim))
    hb_in = jnp.reshape(hb_in_ref[...], (nh * ts, hdim))
    xif_ref[...] = jnp.reshape(
        jnp.dot(hf_in, wif_ref[...], preferred_element_type=jnp.float32)
        + bf_ref[...], (nh, ts, 4 * hdim))
    xib_ref[...] = jnp.reshape(
        jnp.dot(hb_in, wib_ref[...], preferred_element_type=jnp.float32)
        + bb_ref[...], (nh, ts, 4 * hdim))

    whf = whf_ref[...]
    whb = whb_ref[...]
    hf = state_ref[0:nh, :]
    cf = state_ref[nh:2 * nh, :]
    hb = state_ref[2 * nh:3 * nh, :]
    cb = state_ref[3 * nh:4 * nh, :]

    def step(t, carry):
        hf, cf, hb, cb = carry
        tb = ts - 1 - t
        zf = jnp.reshape(xif_ref[:, pl.ds(t, 1), :], (nh, 4 * hdim)) + jnp.dot(
            hf, whf, preferred_element_type=jnp.float32)
        zb = jnp.reshape(xib_ref[:, pl.ds(tb, 1), :], (nh, 4 * hdim)) + jnp.dot(
            hb, whb, preferred_element_type=jnp.float32)
        z = jnp.concatenate([zf, zb], axis=0)
        sig_if = jax.nn.sigmoid(z[:, :2 * hdim])
        g = jnp.tanh(z[:, 2 * hdim:3 * hdim])
        o = jax.nn.sigmoid(z[:, 3 * hdim:])
        cc = jnp.concatenate([cf, cb], axis=0)
        cc = sig_if[:, hdim:] * cc + sig_if[:, :hdim] * g
        hh = o * jnp.tanh(cc)
        hf = hh[0:nh]
        hb = hh[nh:]
        cf = cc[0:nh]
        cb = cc[nh:]
        hf_out_ref[:, pl.ds(t, 1), :] = jnp.reshape(hf, (nh, 1, hdim))
        hb_out_ref[:, pl.ds(tb, 1), :] = jnp.reshape(hb, (nh, 1, hdim))
        return hf, cf, hb, cb

    hf, cf, hb, cb = jax.lax.fori_loop(0, ts, step, (hf, cf, hb, cb),
                                       unroll=2)
    state_ref[0:nh, :] = hf
    state_ref[nh:2 * nh, :] = cf
    state_ref[2 * nh:3 * nh, :] = hb
    state_ref[3 * nh:4 * nh, :] = cb


def _k4_body(nheads, hf_ref, hb_ref, wout_ref, who_ref, stats_ref, st_ref):
    r = pl.program_id(0)
    nr = pl.num_programs(0)
    hdim = hf_ref.shape[2]
    tr = hf_ref.shape[1]

    @pl.when(r == 0)
    def _init():
        st_ref[...] = jnp.concatenate(
            [jnp.full((1, 1), -jnp.inf, jnp.float32),
             jnp.full((1, 1), jnp.inf, jnp.float32),
             jnp.zeros((1, 2), jnp.float32)], axis=1)

    acc = jnp.zeros((tr, wout_ref.shape[1]), jnp.float32)
    for h in range(nheads):
        y = _elu(hf_ref[h] + hb_ref[h])
        acc = acc + jnp.dot(y, wout_ref[h * hdim:(h + 1) * hdim, :],
                            preferred_element_type=jnp.float32)
    who_ref[...] = acc
    st_ref[:, 0:1] = jnp.maximum(st_ref[:, 0:1], jnp.max(acc, keepdims=True))
    st_ref[:, 1:2] = jnp.minimum(st_ref[:, 1:2], jnp.min(acc, keepdims=True))
    st_ref[:, 2:3] = st_ref[:, 2:3] + jnp.sum(acc, keepdims=True)

    @pl.when(r == nr - 1)
    def _fin():
        stats_ref[...] = st_ref[...]


def _k5_body(n, tc, adj_ref, whor_ref, whoc_ref, aout_ref, stats_ref,
             out_ref, acc_ref, l_ref):
    c = pl.program_id(1)
    nc = pl.num_programs(1)

    @pl.when(c == 0)
    def _init():
        l_ref[...] = jnp.zeros_like(l_ref)
        acc_ref[...] = jnp.zeros_like(acc_ref)

    base = c * tc
    col_ok_row = base + jax.lax.broadcasted_iota(jnp.int32, (1, tc), 1) < n
    col_ok_col = base + jax.lax.broadcasted_iota(jnp.int32, (tc, 1), 0) < n
    a = aout_ref[...]
    a0 = a[:, 0:1]
    a1 = a[:, 1:2]
    f2max = jnp.maximum(a1 * stats_ref[:, 0:1], a1 * stats_ref[:, 1:2])
    whoc = jnp.where(col_ok_col, whoc_ref[...], 0.0)
    whoc_row = jnp.reshape(whoc, (1, whoc.shape[0]))
    f1 = whor_ref[...] * a0
    f2 = whoc_row * a1
    m = _leaky(f1 + f2max)
    mask = (adj_ref[...] > 0) & col_ok_row
    p = jnp.where(mask, jnp.exp(_leaky(f1 + f2) - m), 0.0)
    l_ref[...] = l_ref[...] + jnp.sum(p, axis=1, keepdims=True)
    acc_ref[...] = acc_ref[...] + jnp.sum(
        p * whoc_row, axis=1, keepdims=True)

    @pl.when(c == nc - 1)
    def _fin():
        l = l_ref[...]
        mean = stats_ref[:, 2:3] / n
        h_out = jnp.where(l > 0, acc_ref[...] / jnp.maximum(l, 1e-37), mean)
        out_ref[...] = _elu(h_out)


def kernel(x, adj, W_att, a_att, Wi_f, Wh_f, b_f, Wi_b, Wh_b, b_b, W_out,
           a_out):
    n, nfeat = x.shape
    nheads, _, hdim = W_att.shape
    outd = W_out.shape[1]
    f32 = jnp.float32

    trp = _pick_tile(n, 2000)
    tr = _pick_tile(n, 400)
    tc = min(2048, ((n + 127) // 128) * 128)
    ts = _pick_tile(n, 1000)
    ncol = pl.cdiv(n, tc)

    wh, f1, f2, fmax, wmean = pl.pallas_call(
        functools.partial(_k1_body, nheads),
        grid=(n // trp,),
        in_specs=[
            pl.BlockSpec((trp, nfeat), lambda r: (r, 0)),
            pl.BlockSpec((nheads, nfeat, hdim), lambda r: (0, 0, 0)),
            pl.BlockSpec((nheads, 2 * hdim, 1), lambda r: (0, 0, 0)),
        ],
        out_specs=[
            pl.BlockSpec((nheads, trp, hdim), lambda r: (0, r, 0)),
            pl.BlockSpec((trp, nheads), lambda r: (r, 0)),
            pl.BlockSpec((trp, nheads), lambda r: (r, 0)),
            pl.BlockSpec((1, nheads), lambda r: (0, 0)),
            pl.BlockSpec((nheads, hdim), lambda r: (0, 0)),
        ],
        out_shape=[
            jax.ShapeDtypeStruct((nheads, n, hdim), f32),
            jax.ShapeDtypeStruct((n, nheads), f32),
            jax.ShapeDtypeStruct((n, nheads), f32),
            jax.ShapeDtypeStruct((1, nheads), f32),
            jax.ShapeDtypeStruct((nheads, hdim), f32),
        ],
        scratch_shapes=[
            pltpu.VMEM((1, nheads), f32),
            pltpu.VMEM((nheads, hdim), f32),
        ],
    )(x, W_att, a_att)

    hagg = pl.pallas_call(
        functools.partial(_k2_body, nheads, n, tc),
        grid=(n // tr, ncol),
        in_specs=[
            pl.BlockSpec((tr, tc), lambda r, c: (r, c)),
            pl.BlockSpec((tr, nheads), lambda r, c: (r, 0)),
            pl.BlockSpec((tc, nheads), lambda r, c: (c, 0)),
            pl.BlockSpec((nheads, tc, hdim), lambda r, c: (0, c, 0)),
            pl.BlockSpec((1, nheads), lambda r, c: (0, 0)),
            pl.BlockSpec((nheads, hdim), lambda r, c: (0, 0)),
        ],
        out_specs=pl.BlockSpec((nheads, tr, hdim), lambda r, c: (0, r, 0)),
        out_shape=jax.ShapeDtypeStruct((nheads, n, hdim), f32),
        scratch_shapes=[
            pltpu.VMEM((nheads, tr, hdim), f32),
            pltpu.VMEM((tr, nheads), f32),
        ],
    )(adj, f1, f2, wh, fmax, wmean)

    nk = n // ts
    bf2 = jnp.reshape(b_f, (1, 4 * hdim))
    bb2 = jnp.reshape(b_b, (1, 4 * hdim))
    hf, hb = pl.pallas_call(
        functools.partial(_k3_body, nheads, ts),
        grid=(nk,),
        in_specs=[
            pl.BlockSpec((nheads, ts, hdim), lambda k: (0, k, 0)),
            pl.BlockSpec((nheads, ts, hdim), lambda k: (0, nk - 1 - k, 0)),
            pl.BlockSpec((hdim, 4 * hdim), lambda k: (0, 0)),
            pl.BlockSpec((hdim, 4 * hdim), lambda k: (0, 0)),
            pl.BlockSpec((1, 4 * hdim), lambda k: (0, 0)),
            pl.BlockSpec((hdim, 4 * hdim), lambda k: (0, 0)),
            pl.BlockSpec((hdim, 4 * hdim), lambda k: (0, 0)),
            pl.BlockSpec((1, 4 * hdim), lambda k: (0, 0)),
        ],
        out_specs=[
            pl.BlockSpec((nheads, ts, hdim), lambda k: (0, k, 0)),
            pl.BlockSpec((nheads, ts, hdim), lambda k: (0, nk - 1 - k, 0)),
        ],
        out_shape=[
            jax.ShapeDtypeStruct((nheads, n, hdim), f32),
            jax.ShapeDtypeStruct((nheads, n, hdim), f32),
        ],
        scratch_shapes=[
            pltpu.VMEM((nheads, ts, 4 * hdim), f32),
            pltpu.VMEM((nheads, ts, 4 * hdim), f32),
            pltpu.VMEM((4 * nheads, hdim), f32),
        ],
    )(hagg, hagg, Wi_f, Wh_f, bf2, Wi_b, Wh_b, bb2)

    who, stats = pl.pallas_call(
        functools.partial(_k4_body, nheads),
        grid=(n // trp,),
        in_specs=[
            pl.BlockSpec((nheads, trp, hdim), lambda r: (0, r, 0)),
            pl.BlockSpec((nheads, trp, hdim), lambda r: (0, r, 0)),
            pl.BlockSpec((nheads * hdim, outd), lambda r: (0, 0)),
        ],
        out_specs=[
            pl.BlockSpec((trp, outd), lambda r: (r, 0)),
            pl.BlockSpec((1, 4), lambda r: (0, 0)),
        ],
        out_shape=[
            jax.ShapeDtypeStruct((n, outd), f32),
            jax.ShapeDtypeStruct((1, 4), f32),
        ],
        scratch_shapes=[pltpu.VMEM((1, 4), f32)],
    )(hf, hb, W_out)

    aout2 = jnp.reshape(a_out, (1, 2 * outd))
    out = pl.pallas_call(
        functools.partial(_k5_body, n, tc),
        grid=(n // tr, ncol),
        in_specs=[
            pl.BlockSpec((tr, tc), lambda r, c: (r, c)),
            pl.BlockSpec((tr, outd), lambda r, c: (r, 0)),
            pl.BlockSpec((tc, outd), lambda r, c: (c, 0)),
            pl.BlockSpec((1, 2 * outd), lambda r, c: (0, 0)),
            pl.BlockSpec((1, 4), lambda r, c: (0, 0)),
        ],
        out_specs=pl.BlockSpec((tr, outd), lambda r, c: (r, 0)),
        out_shape=jax.ShapeDtypeStruct((n, outd), f32),
        scratch_shapes=[
            pltpu.VMEM((tr, outd), f32),
            pltpu.VMEM((tr, outd), f32),
        ],
    )(adj, who, who, aout2, stats)

    return out
